# Initial kernel scaffold; baseline (speedup 1.0000x reference)
import jax
import jax.numpy as jnp
from jax.experimental import pallas as pl


def kernel(x, edge_index, W, b):
    raise NotImplementedError("write your pallas kernel here")


# speedup vs baseline: 20.5567x; 20.5567x over previous
import functools

import jax
import jax.numpy as jnp
from jax import lax
from jax.experimental import pallas as pl
from jax.experimental.pallas import tpu as pltpu
from jax.experimental.pallas import tpu_sc as plsc

N = 10000
E = 320000
D = 128

NC = 2
NS = 16
NW = NC * NS

CHUNK = 128
NCHUNK = -(-E // (NW * CHUNK))
EDGES_PER_TILE = NCHUNK * CHUNK
E_PAD = NW * EDGES_PER_TILE

N_PAD = 10240
ROWS_PER_TILE = N_PAD // NS
PAD_NODE = N

GRID = 10
BLK = N_PAD // GRID

_mesh = plsc.VectorSubcoreMesh(core_axis_name="c", subcore_axis_name="s")


def _deg_body(dst3, zeros1, degp, idx_v, ones_v, deg_sh):
    c = lax.axis_index("c")
    s = lax.axis_index("s")
    w = c * NS + s
    r0 = s * ROWS_PER_TILE
    pltpu.sync_copy(zeros1.at[pl.ds(r0, ROWS_PER_TILE)],
                    deg_sh.at[pl.ds(r0, ROWS_PER_TILE)])
    pltpu.sync_copy(dst3.at[w], idx_v)
    for i in range(CHUNK // 16):
        ones_v[pl.ds(16 * i, 16)] = jnp.full((16,), 1.0, jnp.float32)
    plsc.subcore_barrier()

    def body(j, carry):
        pltpu.sync_copy(ones_v, deg_sh.at[idx_v.at[j]], add=True)
        return carry

    lax.fori_loop(0, NCHUNK, body, 0)
    plsc.subcore_barrier()

    @pl.when(s == 0)
    def _():
        pltpu.sync_copy(deg_sh, degp.at[c])


_deg_kernel = functools.partial(
    pl.kernel,
    out_type=jax.ShapeDtypeStruct((NC, N_PAD), jnp.float32),
    mesh=_mesh,
    scratch_types=[
        pltpu.VMEM((NCHUNK, CHUNK), jnp.int32),
        pltpu.VMEM((CHUNK,), jnp.float32),
        pltpu.VMEM_SHARED((N_PAD,), jnp.float32),
    ],
)(_deg_body)


def _scat_body(src3, dst3, hp, zrows, accp, srcv, dstv, rows_v, acc_sh, sem):
    c = lax.axis_index("c")
    s = lax.axis_index("s")
    w = c * NS + s
    r0 = s * ROWS_PER_TILE
    pltpu.sync_copy(zrows.at[pl.ds(r0, ROWS_PER_TILE)],
                    acc_sh.at[pl.ds(r0, ROWS_PER_TILE)])
    pltpu.sync_copy(src3.at[w], srcv)
    pltpu.sync_copy(dst3.at[w], dstv)
    plsc.subcore_barrier()

    def body(j, carry):
        pltpu.async_copy(hp.at[srcv.at[j]], rows_v, sem).wait()
        pltpu.sync_copy(rows_v, acc_sh.at[dstv.at[j]], add=True)
        return carry

    lax.fori_loop(0, NCHUNK, body, 0)
    plsc.subcore_barrier()

    @pl.when(s == 0)
    def _():
        pltpu.sync_copy(acc_sh, accp.at[c])


_scat_kernel = functools.partial(
    pl.kernel,
    out_type=jax.ShapeDtypeStruct((NC, N_PAD, D), jnp.float32),
    mesh=_mesh,
    scratch_types=[
        pltpu.VMEM((NCHUNK, CHUNK), jnp.int32),
        pltpu.VMEM((NCHUNK, CHUNK), jnp.int32),
        pltpu.VMEM((CHUNK, D), jnp.float32),
        pltpu.VMEM_SHARED((N_PAD, D), jnp.float32),
        pltpu.SemaphoreType.DMA,
    ],
)(_scat_body)


def _lin_body(x_ref, w_ref, degp_ref, hp_ref):
    deg = 1.0 + degp_ref[0, :] + degp_ref[1, :]
    dinv = lax.rsqrt(deg)[:, None]
    h = jnp.dot(x_ref[...], w_ref[...], preferred_element_type=jnp.float32)
    hp_ref[...] = h * dinv


def _comb_body(accp_ref, hp_ref, degp_ref, b_ref, out_ref):
    deg = 1.0 + degp_ref[0, :] + degp_ref[1, :]
    dinv = lax.rsqrt(deg)[:, None]
    out_ref[...] = dinv * (accp_ref[0] + accp_ref[1] + hp_ref[...]) + b_ref[...]


def kernel(x, edge_index, W, b):
    src = edge_index[0].astype(jnp.int32)
    dst = edge_index[1].astype(jnp.int32)
    npad = E_PAD - E
    pad = jnp.full((npad,), PAD_NODE, jnp.int32)
    src3 = jnp.concatenate([src, pad]).reshape(NW, NCHUNK, CHUNK)
    dst3 = jnp.concatenate([dst, pad]).reshape(NW, NCHUNK, CHUNK)
    x_pad = jnp.zeros((N_PAD, D), jnp.float32).at[:N].set(x)
    zeros1 = jnp.zeros((N_PAD,), jnp.float32)
    zrows = jnp.zeros((N_PAD, D), jnp.float32)

    degp = _deg_kernel(dst3, zeros1)

    hp = pl.pallas_call(
        _lin_body,
        grid=(GRID,),
        in_specs=[
            pl.BlockSpec((BLK, D), lambda i: (i, 0)),
            pl.BlockSpec((D, D), lambda i: (0, 0)),
            pl.BlockSpec((NC, BLK), lambda i: (0, i)),
        ],
        out_specs=pl.BlockSpec((BLK, D), lambda i: (i, 0)),
        out_shape=jax.ShapeDtypeStruct((N_PAD, D), jnp.float32),
    )(x_pad, W, degp)

    accp = _scat_kernel(src3, dst3, hp, zrows)

    out = pl.pallas_call(
        _comb_body,
        grid=(GRID,),
        in_specs=[
            pl.BlockSpec((NC, BLK, D), lambda i: (0, i, 0)),
            pl.BlockSpec((BLK, D), lambda i: (i, 0)),
            pl.BlockSpec((NC, BLK), lambda i: (0, i)),
            pl.BlockSpec((1, D), lambda i: (0, 0)),
        ],
        out_specs=pl.BlockSpec((BLK, D), lambda i: (i, 0)),
        out_shape=jax.ShapeDtypeStruct((N_PAD, D), jnp.float32),
    )(accp, hp, degp, b.reshape(1, D))

    return out[:N]


# speedup vs baseline: 24.0343x; 1.1692x over previous
import functools

import jax
import jax.numpy as jnp
from jax import lax
from jax.experimental import pallas as pl
from jax.experimental.pallas import tpu as pltpu
from jax.experimental.pallas import tpu_sc as plsc

N = 10000
E = 320000
D = 128

NC = 2
NS = 16
NW = NC * NS

CHUNK = 128
NCHUNK = -(-E // (NW * CHUNK))
EDGES_PER_TILE = NCHUNK * CHUNK
E_PAD = NW * EDGES_PER_TILE

N_PAD = 10240
ROWS_PER_TILE = N_PAD // NS
PAD_NODE = N

GRID = 10
BLK = N_PAD // GRID

_mesh = plsc.VectorSubcoreMesh(core_axis_name="c", subcore_axis_name="s")


def _deg_body(dst3, zeros1, degp, idx_v, ones_v, deg_sh):
    c = lax.axis_index("c")
    s = lax.axis_index("s")
    w = c * NS + s
    r0 = s * ROWS_PER_TILE
    pltpu.sync_copy(zeros1.at[pl.ds(r0, ROWS_PER_TILE)],
                    deg_sh.at[pl.ds(r0, ROWS_PER_TILE)])
    pltpu.sync_copy(dst3.at[w], idx_v)
    for i in range(CHUNK // 16):
        ones_v[pl.ds(16 * i, 16)] = jnp.full((16,), 1.0, jnp.float32)
    plsc.subcore_barrier()

    def body(j, carry):
        pltpu.sync_copy(ones_v, deg_sh.at[idx_v.at[j]], add=True)
        return carry

    lax.fori_loop(0, NCHUNK, body, 0)
    plsc.subcore_barrier()

    @pl.when(s == 0)
    def _():
        pltpu.sync_copy(deg_sh, degp.at[c])


_deg_kernel = functools.partial(
    pl.kernel,
    out_type=jax.ShapeDtypeStruct((NC, N_PAD), jnp.float32),
    mesh=_mesh,
    scratch_types=[
        pltpu.VMEM((NCHUNK, CHUNK), jnp.int32),
        pltpu.VMEM((CHUNK,), jnp.float32),
        pltpu.VMEM_SHARED((N_PAD,), jnp.float32),
    ],
)(_deg_body)


def _scat_body(edges3, hp, zrows, accp, idxr, rows_v, acc_sh, sem_i, sem_g):
    c = lax.axis_index("c")
    s = lax.axis_index("s")
    w = c * NS + s
    r0 = s * ROWS_PER_TILE
    pltpu.sync_copy(zrows.at[pl.ds(r0, ROWS_PER_TILE)],
                    acc_sh.at[pl.ds(r0, ROWS_PER_TILE)])
    plsc.subcore_barrier()

    pltpu.sync_copy(edges3.at[w, 0], idxr.at[0])
    pltpu.async_copy(edges3.at[w, 1], idxr.at[1], sem_i)
    pltpu.async_copy(hp.at[idxr.at[0, 0]], rows_v.at[0], sem_g)

    def body(j, carry):
        nxt = j + 1
        cur = lax.rem(j, 2)
        opp = lax.rem(nxt, 2)

        @pl.when(nxt < NCHUNK)
        def _():
            pltpu.make_async_copy(edges3.at[w, nxt], idxr.at[opp],
                                  sem_i).wait()
            pltpu.async_copy(hp.at[idxr.at[opp, 0]], rows_v.at[opp], sem_g)

        pltpu.make_async_copy(hp.at[idxr.at[cur, 0]], rows_v.at[cur],
                              sem_g).wait()
        pltpu.sync_copy(rows_v.at[cur], acc_sh.at[idxr.at[cur, 1]], add=True)

        @pl.when(j + 2 < NCHUNK)
        def _():
            pltpu.async_copy(edges3.at[w, j + 2], idxr.at[cur], sem_i)

        return carry

    lax.fori_loop(0, NCHUNK, body, 0)
    plsc.subcore_barrier()
    pltpu.sync_copy(acc_sh.at[pl.ds(r0, ROWS_PER_TILE)],
                    accp.at[c, pl.ds(r0, ROWS_PER_TILE)])


_scat_kernel = functools.partial(
    pl.kernel,
    out_type=jax.ShapeDtypeStruct((NC, N_PAD, D), jnp.float32),
    mesh=_mesh,
    scratch_types=[
        pltpu.VMEM((2, 2, CHUNK), jnp.int32),
        pltpu.VMEM((2, CHUNK, D), jnp.float32),
        pltpu.VMEM_SHARED((N_PAD, D), jnp.float32),
        pltpu.SemaphoreType.DMA,
        pltpu.SemaphoreType.DMA,
    ],
)(_scat_body)


def _lin_body(x_ref, w_ref, degp_ref, hp_ref):
    deg = 1.0 + degp_ref[0, :] + degp_ref[1, :]
    dinv = lax.rsqrt(deg)[:, None]
    h = jnp.dot(x_ref[...], w_ref[...], preferred_element_type=jnp.float32)
    hp_ref[...] = h * dinv


def _comb_body(accp_ref, hp_ref, degp_ref, b_ref, out_ref):
    deg = 1.0 + degp_ref[0, :] + degp_ref[1, :]
    dinv = lax.rsqrt(deg)[:, None]
    out_ref[...] = dinv * (accp_ref[0] + accp_ref[1] + hp_ref[...]) + b_ref[...]


def kernel(x, edge_index, W, b):
    src = edge_index[0].astype(jnp.int32)
    dst = edge_index[1].astype(jnp.int32)
    npad = E_PAD - E
    pad = jnp.full((npad,), PAD_NODE, jnp.int32)
    src3 = jnp.concatenate([src, pad]).reshape(NW, NCHUNK, CHUNK)
    dst3 = jnp.concatenate([dst, pad]).reshape(NW, NCHUNK, CHUNK)
    edges3 = jnp.stack([src3, dst3], axis=2)
    x_pad = jnp.zeros((N_PAD, D), jnp.float32).at[:N].set(x)
    zeros1 = jnp.zeros((N_PAD,), jnp.float32)
    zrows = jnp.zeros((N_PAD, D), jnp.float32)

    degp = _deg_kernel(dst3, zeros1)

    hp = pl.pallas_call(
        _lin_body,
        grid=(GRID,),
        in_specs=[
            pl.BlockSpec((BLK, D), lambda i: (i, 0)),
            pl.BlockSpec((D, D), lambda i: (0, 0)),
            pl.BlockSpec((NC, BLK), lambda i: (0, i)),
        ],
        out_specs=pl.BlockSpec((BLK, D), lambda i: (i, 0)),
        out_shape=jax.ShapeDtypeStruct((N_PAD, D), jnp.float32),
    )(x_pad, W, degp)

    accp = _scat_kernel(edges3, hp, zrows)

    out = pl.pallas_call(
        _comb_body,
        grid=(GRID,),
        in_specs=[
            pl.BlockSpec((NC, BLK, D), lambda i: (0, i, 0)),
            pl.BlockSpec((BLK, D), lambda i: (i, 0)),
            pl.BlockSpec((NC, BLK), lambda i: (0, i)),
            pl.BlockSpec((1, D), lambda i: (0, 0)),
        ],
        out_specs=pl.BlockSpec((BLK, D), lambda i: (i, 0)),
        out_shape=jax.ShapeDtypeStruct((N_PAD, D), jnp.float32),
    )(accp, hp, degp, b.reshape(1, D))

    return out[:N]


# speedup vs baseline: 27.2535x; 1.1339x over previous
import functools

import jax
import jax.numpy as jnp
from jax import lax
from jax.experimental import pallas as pl
from jax.experimental.pallas import tpu as pltpu
from jax.experimental.pallas import tpu_sc as plsc

N = 10000
E = 320000
D = 128

NC = 2
NS = 16
NW = NC * NS

CHUNK = 128
NCH0 = 54
NCH1 = 103
TOT_CHUNKS = NS * (NCH0 + NCH1)
E_PAD = TOT_CHUNKS * CHUNK
PAD_NODE = N

N_PAD = 10240
ROWS_PER_TILE = N_PAD // NS
GRID = 10
BLK = N_PAD // GRID

_mesh = plsc.VectorSubcoreMesh(core_axis_name="c", subcore_axis_name="s")


def _tile_span(c, s):
    base = jnp.where(c == 0, s * NCH0, NS * NCH0 + s * NCH1)
    nch = jnp.where(c == 0, NCH0, NCH1)
    return base, nch


def _deg_body(edges3, degp, st_v, ones_v, zdeg_v, deg_sh, sem_s):
    c = lax.axis_index("c")
    s = lax.axis_index("s")
    base, nch = _tile_span(c, s)
    r0 = s * ROWS_PER_TILE

    z16 = jnp.zeros((16,), jnp.float32)
    for k in range(ROWS_PER_TILE // 16):
        zdeg_v[pl.ds(16 * k, 16)] = z16
    for k in range(CHUNK // 16):
        ones_v[pl.ds(16 * k, 16)] = jnp.full((16,), 1.0, jnp.float32)
    pltpu.sync_copy(zdeg_v, deg_sh.at[pl.ds(r0, ROWS_PER_TILE)])
    pltpu.sync_copy(edges3.at[pl.ds(base, NCH1)], st_v)
    plsc.subcore_barrier()

    WIN = 8

    def roll(j, carry):
        @pl.when(j < nch)
        def _():
            pltpu.async_copy(ones_v, deg_sh.at[st_v.at[j, 1]], sem_s,
                             add=True)

        @pl.when((j >= WIN) & (j - WIN < nch))
        def _():
            pltpu.make_async_copy(ones_v, deg_sh.at[st_v.at[0, 1]],
                                  sem_s).wait()

        return carry

    lax.fori_loop(0, NCH1 + WIN, roll, 0)
    plsc.subcore_barrier()
    pltpu.sync_copy(deg_sh.at[pl.ds(r0, ROWS_PER_TILE)],
                    degp.at[c, pl.ds(r0, ROWS_PER_TILE)])


_deg_kernel = functools.partial(
    pl.kernel,
    out_type=jax.ShapeDtypeStruct((NC, N_PAD), jnp.float32),
    mesh=_mesh,
    scratch_types=[
        pltpu.VMEM((NCH1, 2, CHUNK), jnp.int32),
        pltpu.VMEM((CHUNK,), jnp.float32),
        pltpu.VMEM((ROWS_PER_TILE,), jnp.float32),
        pltpu.VMEM_SHARED((N_PAD,), jnp.float32),
        pltpu.SemaphoreType.DMA,
    ],
)(_deg_body)


def _scat_body(edges3, hp, accp, idxr, rows_v, acc_sh, sem_i, sem_g):
    c = lax.axis_index("c")
    s = lax.axis_index("s")
    base, nch = _tile_span(c, s)
    r0 = s * ROWS_PER_TILE

    z16 = jnp.zeros((16,), jnp.float32)
    for i in range(CHUNK):
        for k in range(D // 16):
            rows_v[0, i, pl.ds(16 * k, 16)] = z16
    for t in range(ROWS_PER_TILE // CHUNK):
        pltpu.sync_copy(rows_v.at[0],
                        acc_sh.at[pl.ds(r0 + t * CHUNK, CHUNK)])
    plsc.subcore_barrier()

    pltpu.sync_copy(edges3.at[base], idxr.at[0])
    pltpu.async_copy(edges3.at[base + 1], idxr.at[1], sem_i)
    pltpu.async_copy(hp.at[idxr.at[0, 0]], rows_v.at[0], sem_g)

    def body(j, carry):
        @pl.when(j < nch)
        def _():
            nxt = j + 1
            cur = lax.rem(j, 2)
            opp = lax.rem(nxt, 2)

            @pl.when(nxt < nch)
            def _():
                pltpu.make_async_copy(edges3.at[base + nxt], idxr.at[opp],
                                      sem_i).wait()
                pltpu.async_copy(hp.at[idxr.at[opp, 0]], rows_v.at[opp],
                                 sem_g)

            pltpu.make_async_copy(hp.at[idxr.at[cur, 0]], rows_v.at[cur],
                                  sem_g).wait()
            pltpu.sync_copy(rows_v.at[cur], acc_sh.at[idxr.at[cur, 1]],
                            add=True)

            @pl.when(j + 2 < nch)
            def _():
                pltpu.async_copy(edges3.at[base + j + 2], idxr.at[cur],
                                 sem_i)

        return carry

    lax.fori_loop(0, NCH1, body, 0)
    plsc.subcore_barrier()
    pltpu.sync_copy(acc_sh.at[pl.ds(r0, ROWS_PER_TILE)],
                    accp.at[c, pl.ds(r0, ROWS_PER_TILE)])


_scat_kernel = functools.partial(
    pl.kernel,
    out_type=jax.ShapeDtypeStruct((NC, N_PAD, D), jnp.float32),
    mesh=_mesh,
    scratch_types=[
        pltpu.VMEM((2, 2, CHUNK), jnp.int32),
        pltpu.VMEM((2, CHUNK, D), jnp.float32),
        pltpu.VMEM_SHARED((N_PAD, D), jnp.float32),
        pltpu.SemaphoreType.DMA,
        pltpu.SemaphoreType.DMA,
    ],
)(_scat_body)


def _lin_body(x_ref, w_ref, degp_ref, hp_ref):
    deg = 1.0 + degp_ref[0, :] + degp_ref[1, :]
    dinv = lax.rsqrt(deg)[:, None]
    h = jnp.dot(x_ref[...], w_ref[...], preferred_element_type=jnp.float32)
    hp_ref[...] = h * dinv


def _comb_body(accp_ref, hp_ref, degp_ref, b_ref, out_ref):
    deg = 1.0 + degp_ref[0, :] + degp_ref[1, :]
    dinv = lax.rsqrt(deg)[:, None]
    out_ref[...] = dinv * (accp_ref[0] + accp_ref[1] + hp_ref[...]) + b_ref[...]


def kernel(x, edge_index, W, b):
    src = edge_index[0].astype(jnp.int32)
    dst = edge_index[1].astype(jnp.int32)
    npad = E_PAD - E
    pad = jnp.full((npad,), PAD_NODE, jnp.int32)
    srcp = jnp.concatenate([src, pad]).reshape(TOT_CHUNKS, CHUNK)
    dstp = jnp.concatenate([dst, pad]).reshape(TOT_CHUNKS, CHUNK)
    edges3 = jnp.stack([srcp, dstp], axis=1)
    x_pad = jnp.zeros((N_PAD, D), jnp.float32).at[:N].set(x)

    degp = _deg_kernel(edges3)

    hp = pl.pallas_call(
        _lin_body,
        grid=(GRID,),
        in_specs=[
            pl.BlockSpec((BLK, D), lambda i: (i, 0)),
            pl.BlockSpec((D, D), lambda i: (0, 0)),
            pl.BlockSpec((NC, BLK), lambda i: (0, i)),
        ],
        out_specs=pl.BlockSpec((BLK, D), lambda i: (i, 0)),
        out_shape=jax.ShapeDtypeStruct((N_PAD, D), jnp.float32),
    )(x_pad, W, degp)

    accp = _scat_kernel(edges3, hp)

    out = pl.pallas_call(
        _comb_body,
        grid=(GRID,),
        in_specs=[
            pl.BlockSpec((NC, BLK, D), lambda i: (0, i, 0)),
            pl.BlockSpec((BLK, D), lambda i: (i, 0)),
            pl.BlockSpec((NC, BLK), lambda i: (0, i)),
            pl.BlockSpec((1, D), lambda i: (0, 0)),
        ],
        out_specs=pl.BlockSpec((BLK, D), lambda i: (i, 0)),
        out_shape=jax.ShapeDtypeStruct((N_PAD, D), jnp.float32),
    )(accp, hp, degp, b.reshape(1, D))

    return out[:N]


# speedup vs baseline: 37.8936x; 1.3904x over previous
import functools

import jax
import jax.numpy as jnp
from jax import lax
from jax.experimental import pallas as pl
from jax.experimental.pallas import tpu as pltpu
from jax.experimental.pallas import tpu_sc as plsc

N = 10000
E = 320000
D = 128

NC = 2
NS = 16
NW = NC * NS

CHUNK = 128
TOT_CHUNKS = E // CHUNK
NCHF = 103
NCHS = 53
REM = TOT_CHUNKS - NS * (NCHF + NCHS)
FAST_TOT = NS * NCHF + REM
MAXCH = NCHF + 1

N_PAD = 10240
ROWS_PER_TILE = N_PAD // NS
GRID = 10
BLK = N_PAD // GRID

_mesh = plsc.VectorSubcoreMesh(core_axis_name="c", subcore_axis_name="s")


def _tile_span(c, s):
    base0 = s * NCHF + jnp.minimum(s, REM)
    nch0 = NCHF + jnp.where(s < REM, 1, 0)
    base1 = FAST_TOT + s * NCHS
    base = jnp.where(c == 0, base0, base1)
    nch = jnp.where(c == 0, nch0, NCHS)
    return base, nch


def _deg_body(edges3, degp, st_v, ones_v, zdeg_v, deg_sh, sem_s):
    c = lax.axis_index("c")
    s = lax.axis_index("s")
    base, nch = _tile_span(c, s)
    r0 = s * ROWS_PER_TILE

    z16 = jnp.zeros((16,), jnp.float32)
    for k in range(ROWS_PER_TILE // 16):
        zdeg_v[pl.ds(16 * k, 16)] = z16
    for k in range(CHUNK // 16):
        ones_v[pl.ds(16 * k, 16)] = jnp.full((16,), 1.0, jnp.float32)
    pltpu.sync_copy(zdeg_v, deg_sh.at[pl.ds(r0, ROWS_PER_TILE)])
    clb = jnp.minimum(base, TOT_CHUNKS - MAXCH)
    off = base - clb
    pltpu.sync_copy(edges3.at[pl.ds(clb, MAXCH)], st_v)
    plsc.subcore_barrier()

    WIN = 8

    def roll(j, carry):
        @pl.when(j < nch)
        def _():
            pltpu.async_copy(ones_v, deg_sh.at[st_v.at[off + j, 1]], sem_s,
                             add=True)

        @pl.when((j >= WIN) & (j - WIN < nch))
        def _():
            pltpu.make_async_copy(ones_v, deg_sh.at[st_v.at[0, 1]],
                                  sem_s).wait()

        return carry

    lax.fori_loop(0, MAXCH + WIN, roll, 0)
    plsc.subcore_barrier()
    pltpu.sync_copy(deg_sh.at[pl.ds(r0, ROWS_PER_TILE)],
                    degp.at[c, pl.ds(r0, ROWS_PER_TILE)])


_deg_kernel = functools.partial(
    pl.kernel,
    out_type=jax.ShapeDtypeStruct((NC, N_PAD), jnp.float32),
    mesh=_mesh,
    scratch_types=[
        pltpu.VMEM((MAXCH, 2, CHUNK), jnp.int32),
        pltpu.VMEM((CHUNK,), jnp.float32),
        pltpu.VMEM((ROWS_PER_TILE,), jnp.float32),
        pltpu.VMEM_SHARED((N_PAD,), jnp.float32),
        pltpu.SemaphoreType.DMA,
    ],
)(_deg_body)


def _scat_body(edges3, hp, accp, idxr, rows_v, acc_sh, sem_i, sem_g):
    c = lax.axis_index("c")
    s = lax.axis_index("s")
    base, nch = _tile_span(c, s)
    r0 = s * ROWS_PER_TILE

    z16 = jnp.zeros((16,), jnp.float32)
    for i in range(CHUNK):
        for k in range(D // 16):
            rows_v[0, i, pl.ds(16 * k, 16)] = z16
    for t in range(ROWS_PER_TILE // CHUNK):
        pltpu.sync_copy(rows_v.at[0],
                        acc_sh.at[pl.ds(r0 + t * CHUNK, CHUNK)])
    plsc.subcore_barrier()

    pltpu.sync_copy(edges3.at[base], idxr.at[0])
    pltpu.async_copy(edges3.at[base + 1], idxr.at[1], sem_i)
    pltpu.async_copy(hp.at[idxr.at[0, 0]], rows_v.at[0], sem_g)

    def body(j, carry):
        @pl.when(j < nch)
        def _():
            nxt = j + 1
            cur = lax.rem(j, 2)
            opp = lax.rem(nxt, 2)

            @pl.when(nxt < nch)
            def _():
                pltpu.make_async_copy(edges3.at[base + nxt], idxr.at[opp],
                                      sem_i).wait()
                pltpu.async_copy(hp.at[idxr.at[opp, 0]], rows_v.at[opp],
                                 sem_g)

            pltpu.make_async_copy(hp.at[idxr.at[cur, 0]], rows_v.at[cur],
                                  sem_g).wait()
            pltpu.sync_copy(rows_v.at[cur], acc_sh.at[idxr.at[cur, 1]],
                            add=True)

            @pl.when(j + 2 < nch)
            def _():
                pltpu.async_copy(edges3.at[base + j + 2], idxr.at[cur],
                                 sem_i)

        return carry

    lax.fori_loop(0, MAXCH, body, 0)
    plsc.subcore_barrier()
    pltpu.sync_copy(acc_sh.at[pl.ds(r0, ROWS_PER_TILE)],
                    accp.at[c, pl.ds(r0, ROWS_PER_TILE)])


_scat_kernel = functools.partial(
    pl.kernel,
    out_type=jax.ShapeDtypeStruct((NC, N_PAD, D), jnp.float32),
    mesh=_mesh,
    scratch_types=[
        pltpu.VMEM((2, 2, CHUNK), jnp.int32),
        pltpu.VMEM((2, CHUNK, D), jnp.float32),
        pltpu.VMEM_SHARED((N_PAD, D), jnp.float32),
        pltpu.SemaphoreType.DMA,
        pltpu.SemaphoreType.DMA,
    ],
)(_scat_body)


def _lin_body(x_ref, w_ref, degp_ref, hp_ref):
    deg = 1.0 + degp_ref[0, :] + degp_ref[1, :]
    dinv = lax.rsqrt(deg)[:, None]
    h = jnp.dot(x_ref[...], w_ref[...], preferred_element_type=jnp.float32)
    hp_ref[...] = h * dinv


def _comb_body(accp_ref, hp_ref, degp_ref, b_ref, out_ref):
    deg = 1.0 + degp_ref[0, :] + degp_ref[1, :]
    dinv = lax.rsqrt(deg)[:, None]
    out_ref[...] = dinv * (accp_ref[0] + accp_ref[1] + hp_ref[...]) + b_ref[...]


def kernel(x, edge_index, W, b):
    ei = edge_index.astype(jnp.int32)
    edges3 = jnp.stack([ei[0].reshape(TOT_CHUNKS, CHUNK),
                        ei[1].reshape(TOT_CHUNKS, CHUNK)], axis=1)

    degp = _deg_kernel(edges3)

    hp = pl.pallas_call(
        _lin_body,
        grid=(GRID,),
        in_specs=[
            pl.BlockSpec((BLK, D), lambda i: (i, 0)),
            pl.BlockSpec((D, D), lambda i: (0, 0)),
            pl.BlockSpec((NC, BLK), lambda i: (0, i)),
        ],
        out_specs=pl.BlockSpec((BLK, D), lambda i: (i, 0)),
        out_shape=jax.ShapeDtypeStruct((N, D), jnp.float32),
    )(x, W, degp)

    accp = _scat_kernel(edges3, hp)

    out = pl.pallas_call(
        _comb_body,
        grid=(GRID,),
        in_specs=[
            pl.BlockSpec((NC, BLK, D), lambda i: (0, i, 0)),
            pl.BlockSpec((BLK, D), lambda i: (i, 0)),
            pl.BlockSpec((NC, BLK), lambda i: (0, i)),
            pl.BlockSpec((1, D), lambda i: (0, 0)),
        ],
        out_specs=pl.BlockSpec((BLK, D), lambda i: (i, 0)),
        out_shape=jax.ShapeDtypeStruct((N, D), jnp.float32),
    )(accp, hp, degp, b.reshape(1, D))

    return out
